# Initial kernel scaffold; baseline (speedup 1.0000x reference)
import jax
import jax.numpy as jnp
from jax.experimental import pallas as pl


def kernel(gate_logits, attention_mask):
    raise NotImplementedError("write your pallas kernel here")


# speedup vs baseline: 19.8858x; 19.8858x over previous
import functools

import jax
import jax.numpy as jnp
from jax import lax
from jax.experimental import pallas as pl
from jax.experimental.pallas import tpu as pltpu
from jax.experimental.pallas import tpu_sc as plsc

_Q_DENOM = 1.00001
_TOP_P = 0.75
_KEEP_TOP_K = 2
_ENT_THRESH = 0.5
_DYN_COEF = 0.01
_AUX_COEF = 0.001

_NC, _NS, _L = 2, 16, 16
_NW = _NC * _NS


def _build_partials(n_tokens, mask_len):
    tpw = n_tokens // _NW
    groups = tpw // _L

    mesh = plsc.VectorSubcoreMesh(
        core_axis_name="c", subcore_axis_name="s",
        num_cores=_NC, num_subcores=_NS,
    )

    @functools.partial(
        pl.kernel,
        out_type=jax.ShapeDtypeStruct((_NW, 4, _L), jnp.float32),
        mesh=mesh,
        scratch_types=[
            pltpu.VMEM((tpw * _L,), jnp.float32),
            pltpu.VMEM((tpw,), jnp.float32),
            pltpu.VMEM((_L,), jnp.float32),
            pltpu.VMEM((_L,), jnp.float32),
            pltpu.VMEM((_L,), jnp.float32),
            pltpu.VMEM((_L,), jnp.float32),
        ],
        compiler_params=pltpu.CompilerParams(needs_layout_passes=False),
    )
    def body(logits_hbm, maskf_hbm, out_hbm, logits_v, mask_v, a_v, b_v, sq_v, m_v):
        wid = lax.axis_index("s") * _NC + lax.axis_index("c")
        base = wid * tpw
        pltpu.sync_copy(logits_hbm.at[pl.ds(base * _L, tpw * _L)], logits_v)
        pltpu.sync_copy(maskf_hbm.at[pl.ds(lax.rem(base, mask_len), tpw)], mask_v)

        zeros = jnp.zeros((_L,), jnp.float32)
        a_v[...] = zeros
        b_v[...] = zeros
        sq_v[...] = zeros
        m_v[...] = zeros

        lane = lax.iota(jnp.int32, _L)
        keep_k = lane < _KEEP_TOP_K
        last = jnp.full((_L,), _L - 1, jnp.int32)

        @pl.loop(0, groups)
        def _(g):
            tbase = g * _L
            mvec = mask_v[pl.ds(tbase, _L)]
            for i in range(_L):
                v = logits_v[pl.ds((tbase + i) * _L, _L)]
                ev = jnp.exp(v)
                skey, sval = plsc.sort_key_val(ev, lane, descending=True)
                c = plsc.cumsum(skey)
                s_spl = jnp.take_along_axis(c, last, axis=0,
                                            mode="promise_in_bounds")
                p = skey / s_spl
                sq = p * p
                csq = plsc.cumsum(sq)
                sqs = jnp.take_along_axis(csq, last, axis=0,
                                          mode="promise_in_bounds")
                ent = (1.0 - sqs) / _Q_DENOM
                keep = ((c <= s_spl * _TOP_P) | keep_k) | (ent > _ENT_THRESH)
                w = jnp.where(keep, p, 0.0)
                mt = jnp.take_along_axis(mvec, jnp.full((_L,), i, jnp.int32),
                                         axis=0, mode="promise_in_bounds")
                plsc.addupdate_scatter(a_v, [sval], w * mt)
                plsc.addupdate_scatter(b_v, [sval], p * mt)
                sq_v[...] = sq_v[...] + sq
            m_v[...] = m_v[...] + mvec

        pltpu.sync_copy(a_v, out_hbm.at[wid, 0])
        pltpu.sync_copy(b_v, out_hbm.at[wid, 1])
        pltpu.sync_copy(sq_v, out_hbm.at[wid, 2])
        pltpu.sync_copy(m_v, out_hbm.at[wid, 3])

    return body


def kernel(gate_logits, attention_mask):
    n_tokens, n_experts = gate_logits.shape
    maskf = attention_mask.reshape(-1).astype(jnp.float32)
    parts = _build_partials(n_tokens, maskf.shape[0])(
        gate_logits.reshape(-1), maskf)
    a = parts[:, 0, :].sum(axis=0)
    b = parts[:, 1, :].sum(axis=0)
    sq_sum = parts[:, 2, :].sum()
    m_sum = parts[:, 3, :].sum()
    denom = m_sum + 1e-8
    entropy_loss = (n_tokens - sq_sum) / _Q_DENOM / n_tokens
    load_balance = n_experts * jnp.sum((a / denom) * (b / denom))
    return _DYN_COEF * entropy_loss + _AUX_COEF * load_balance


# speedup vs baseline: 32.2243x; 1.6205x over previous
import functools

import jax
import jax.numpy as jnp
from jax import lax
from jax.experimental import pallas as pl
from jax.experimental.pallas import tpu as pltpu
from jax.experimental.pallas import tpu_sc as plsc

_Q_DENOM = 1.00001
_TOP_P = 0.75
_KEEP_TOP_K = 2
_ENT_THRESH = 0.5
_DYN_COEF = 0.01
_AUX_COEF = 0.001

_NC, _NS, _L = 2, 16, 16
_NW = _NC * _NS


def _build_partials(n_tokens, mask_len):
    tpw = n_tokens // _NW
    groups = tpw // _L

    mesh = plsc.VectorSubcoreMesh(
        core_axis_name="c", subcore_axis_name="s",
        num_cores=_NC, num_subcores=_NS,
    )

    @functools.partial(
        pl.kernel,
        out_type=jax.ShapeDtypeStruct((_NW, 4, _L), jnp.float32),
        mesh=mesh,
        scratch_types=[
            pltpu.VMEM((tpw * _L,), jnp.float32),
            pltpu.VMEM((tpw,), jnp.float32),
            pltpu.VMEM((_L,), jnp.float32),
            pltpu.VMEM((_L,), jnp.float32),
            pltpu.VMEM((_L,), jnp.float32),
            pltpu.VMEM((_L,), jnp.float32),
        ],
        compiler_params=pltpu.CompilerParams(needs_layout_passes=False),
    )
    def body(logits_hbm, maskf_hbm, out_hbm, logits_v, mask_v, a_v, b_v, sq_v, m_v):
        wid = lax.axis_index("s") * _NC + lax.axis_index("c")
        base = wid * tpw
        pltpu.sync_copy(logits_hbm.at[pl.ds(base * _L, tpw * _L)], logits_v)
        pltpu.sync_copy(maskf_hbm.at[pl.ds(lax.rem(base, mask_len), tpw)], mask_v)

        zeros = jnp.zeros((_L,), jnp.float32)
        a_v[...] = zeros
        b_v[...] = zeros

        lane = lax.iota(jnp.int32, _L)
        keep_k = lane < _KEEP_TOP_K
        last = jnp.full((_L,), _L - 1, jnp.int32)
        bfly = [lane ^ (1 << k) for k in range(4)]

        def lane_sum(x):
            for idx in bfly:
                x = x + jnp.take_along_axis(x, idx, axis=0,
                                            mode="promise_in_bounds")
            return x

        @plsc.parallel_loop(0, groups, carry=(zeros, zeros))
        def _(g, acc):
            sq_acc, m_acc = acc
            tbase = g * _L
            mvec = mask_v[pl.ds(tbase, _L)]
            for i in range(_L):
                v = logits_v[pl.ds((tbase + i) * _L, _L)]
                ev = jnp.exp(v)
                skey, sval = plsc.sort_key_val(ev, lane, descending=True)
                c = plsc.cumsum(skey)
                s_spl = jnp.take_along_axis(c, last, axis=0,
                                            mode="promise_in_bounds")
                p = skey / s_spl
                sq = p * p
                sqs = lane_sum(sq)
                ent = (1.0 - sqs) / _Q_DENOM
                keep = ((c <= s_spl * _TOP_P) | keep_k) | (ent > _ENT_THRESH)
                w = jnp.where(keep, p, 0.0)
                mt = jnp.take_along_axis(mvec, jnp.full((_L,), i, jnp.int32),
                                         axis=0, mode="promise_in_bounds")
                plsc.addupdate_scatter(a_v, [sval], w * mt)
                plsc.addupdate_scatter(b_v, [sval], p * mt)
                sq_acc = sq_acc + sq
            return sq_acc, m_acc + mvec

        sq_acc, m_acc = _
        sq_v[...] = sq_acc
        m_v[...] = m_acc
        pltpu.sync_copy(a_v, out_hbm.at[wid, 0])
        pltpu.sync_copy(b_v, out_hbm.at[wid, 1])
        pltpu.sync_copy(sq_v, out_hbm.at[wid, 2])
        pltpu.sync_copy(m_v, out_hbm.at[wid, 3])

    return body


def kernel(gate_logits, attention_mask):
    n_tokens, n_experts = gate_logits.shape
    maskf = attention_mask.reshape(-1).astype(jnp.float32)
    parts = _build_partials(n_tokens, maskf.shape[0])(
        gate_logits.reshape(-1), maskf)
    a = parts[:, 0, :].sum(axis=0)
    b = parts[:, 1, :].sum(axis=0)
    sq_sum = parts[:, 2, :].sum()
    m_sum = parts[:, 3, :].sum()
    denom = m_sum + 1e-8
    entropy_loss = (n_tokens - sq_sum) / _Q_DENOM / n_tokens
    load_balance = n_experts * jnp.sum((a / denom) * (b / denom))
    return _DYN_COEF * entropy_loss + _AUX_COEF * load_balance


# speedup vs baseline: 32.3934x; 1.0052x over previous
import functools

import jax
import jax.numpy as jnp
from jax import lax
from jax.experimental import pallas as pl
from jax.experimental.pallas import tpu as pltpu
from jax.experimental.pallas import tpu_sc as plsc

_Q_DENOM = 1.00001
_TOP_P = 0.75
_KEEP_TOP_K = 2
_ENT_THRESH = 0.5
_DYN_COEF = 0.01
_AUX_COEF = 0.001

_NC, _NS, _L = 2, 16, 16
_NW = _NC * _NS


def _build_partials(n_tokens, mask_len):
    tpw = n_tokens // _NW
    groups = tpw // _L

    mesh = plsc.VectorSubcoreMesh(
        core_axis_name="c", subcore_axis_name="s",
        num_cores=_NC, num_subcores=_NS,
    )

    @functools.partial(
        pl.kernel,
        out_type=jax.ShapeDtypeStruct((_NW, 4, _L), jnp.float32),
        mesh=mesh,
        scratch_types=[
            pltpu.VMEM((tpw, _L), jnp.float32),
            pltpu.VMEM((tpw,), jnp.float32),
            pltpu.VMEM((_L,), jnp.float32),
            pltpu.VMEM((_L,), jnp.float32),
            pltpu.VMEM((_L,), jnp.float32),
            pltpu.VMEM((_L,), jnp.float32),
        ],
        compiler_params=pltpu.CompilerParams(needs_layout_passes=False,
                                             use_tc_tiling_on_sc=False),
    )
    def body(logits_hbm, maskf_hbm, out_hbm, logits_v, mask_v, a_v, b_v, sq_v, m_v):
        wid = lax.axis_index("s") * _NC + lax.axis_index("c")
        base = wid * tpw
        pltpu.sync_copy(logits_hbm.at[pl.ds(base, tpw), :], logits_v)
        pltpu.sync_copy(maskf_hbm.at[pl.ds(lax.rem(base, mask_len), tpw)], mask_v)

        zeros = jnp.zeros((_L,), jnp.float32)
        a_v[...] = zeros
        b_v[...] = zeros

        lane = lax.iota(jnp.int32, _L)
        keep_k = lane < _KEEP_TOP_K
        last = jnp.full((_L,), _L - 1, jnp.int32)
        bfly = [lane ^ (1 << k) for k in range(4)]

        def lane_sum(x):
            for idx in bfly:
                x = x + jnp.take_along_axis(x, idx, axis=0,
                                            mode="promise_in_bounds")
            return x

        @plsc.parallel_loop(0, groups, carry=(zeros, zeros))
        def _(g, acc):
            sq_acc, m_acc = acc
            tbase = g * _L
            mvec = mask_v[pl.ds(tbase, _L)]
            for i in range(_L):
                v = logits_v[tbase + i]
                ev = jnp.exp(v)
                skey, sval = plsc.sort_key_val(ev, lane, descending=True)
                c = plsc.cumsum(skey)
                s_spl = jnp.take_along_axis(c, last, axis=0,
                                            mode="promise_in_bounds")
                p = skey / s_spl
                sq = p * p
                sqs = lane_sum(sq)
                ent = (1.0 - sqs) / _Q_DENOM
                keep = ((c <= s_spl * _TOP_P) | keep_k) | (ent > _ENT_THRESH)
                w = jnp.where(keep, p, 0.0)
                mt = jnp.take_along_axis(mvec, jnp.full((_L,), i, jnp.int32),
                                         axis=0, mode="promise_in_bounds")
                plsc.addupdate_scatter(a_v, [sval], w * mt)
                plsc.addupdate_scatter(b_v, [sval], p * mt)
                sq_acc = sq_acc + sq
            return sq_acc, m_acc + mvec

        sq_acc, m_acc = _
        sq_v[...] = sq_acc
        m_v[...] = m_acc
        pltpu.sync_copy(a_v, out_hbm.at[wid, 0])
        pltpu.sync_copy(b_v, out_hbm.at[wid, 1])
        pltpu.sync_copy(sq_v, out_hbm.at[wid, 2])
        pltpu.sync_copy(m_v, out_hbm.at[wid, 3])

    return body


def kernel(gate_logits, attention_mask):
    n_tokens, n_experts = gate_logits.shape
    maskf = attention_mask.reshape(-1).astype(jnp.float32)
    parts = _build_partials(n_tokens, maskf.shape[0])(gate_logits, maskf)
    a = parts[:, 0, :].sum(axis=0)
    b = parts[:, 1, :].sum(axis=0)
    sq_sum = parts[:, 2, :].sum()
    m_sum = parts[:, 3, :].sum()
    denom = m_sum + 1e-8
    entropy_loss = (n_tokens - sq_sum) / _Q_DENOM / n_tokens
    load_balance = n_experts * jnp.sum((a / denom) * (b / denom))
    return _DYN_COEF * entropy_loss + _AUX_COEF * load_balance
